# baseline (device time: 57380 ns/iter reference)
import jax
import jax.numpy as jnp
from jax import lax
from jax.experimental import pallas as pl
from jax.experimental.pallas import tpu as pltpu

B, SQ, H, D = 4, 256, 16, 64
ROWS = B * H * D
SCALE = D ** -0.5

C = 16
CH = ROWS // C
CPB = C // B

tn = (((0,), (0,)), ((), ()))
nn = (((1,), (0,)), ((), ()))


def _half_head(qT, kT, vT):
    sT = lax.dot_general(kT, qT, tn, preferred_element_type=jnp.float32) * SCALE
    m = jnp.max(sT, axis=0, keepdims=True)
    p = jnp.exp(sT - m)
    den = jnp.sum(p, axis=0, keepdims=True)
    o = lax.dot_general(vT, p.astype(jnp.bfloat16), nn,
                        preferred_element_type=jnp.float32)
    return o, m, den


def _body(q_ref, kb_ref, vb_ref, o_ref,
          krbuf, vrbuf, olscr, mscr,
          s1, r1, s2, r2, ybar, xbar):
    i = pl.program_id(0)
    my_x = lax.axis_index("x")
    my_y = lax.axis_index("y")
    ynbr = (my_x, 1 - my_y)
    xnbr = (1 - my_x, my_y)

    mk = lambda **kw: pltpu.make_async_remote_copy(
        device_id_type=pl.DeviceIdType.MESH, **kw)
    HPC = H // CPB
    ch = lambda ref, c: ref.at[c // CPB, pl.ds((c % CPB) * HPC, HPC)]

    def mk_p1(c, send_ref, recv1):
        return mk(src_ref=ch(send_ref, c), dst_ref=ch(recv1, c),
                  send_sem=s1.at[c], recv_sem=r1.at[c], device_id=ynbr)

    def mk_fwd(c, recv1):
        return mk(src_ref=ch(recv1, c), dst_ref=ch(recv1, c),
                  send_sem=s2.at[c], recv_sem=r2.at[c], device_id=xnbr)

    def mk_p2w(c, send_ref, recv2):
        return mk(src_ref=ch(send_ref, c), dst_ref=ch(recv2, c),
                  send_sem=s1.at[c], recv_sem=r2.at[c], device_id=xnbr)

    def comm(fn):
        pl.when(my_x == 0)(lambda: fn(kb_ref, krbuf, vrbuf))
        pl.when(my_x == 1)(lambda: fn(vb_ref, vrbuf, krbuf))

    @pl.when(i == 0)
    def _():
        pl.semaphore_signal(ybar, inc=1, device_id=ynbr,
                            device_id_type=pl.DeviceIdType.MESH)
        pl.semaphore_signal(xbar, inc=1, device_id=xnbr,
                            device_id_type=pl.DeviceIdType.MESH)
        pl.semaphore_wait(ybar, 1)
        pl.semaphore_wait(xbar, 1)

        def start_all(send_ref, recv1, recv2):
            for c in range(C):
                mk_p1(c, send_ref, recv1).start()
        comm(start_all)

    @pl.when((i >= 1) & (i <= B))
    def _():
        b = i - 1
        for h in range(H):
            qT = q_ref[0, h].astype(jnp.bfloat16)
            o, m, den = _half_head(qT, kb_ref[b, h], vb_ref[b, h])
            olscr[b, h] = o
            mscr[b, h, 0:1, :] = m
            mscr[b, h, 1:2, :] = den

        def pump(send_ref, recv1, recv2):
            def step(j, carry):
                c = b * CPB + j
                mk_p1(c, send_ref, recv1).wait_recv()
                mk_fwd(c, recv1).start()
                return carry
            lax.fori_loop(0, CPB, step, 0)
        comm(pump)

    @pl.when(i >= B + 1)
    def _():
        b = i - (B + 1)

        def drain(send_ref, recv1, recv2):
            def step(j, carry):
                mk_p2w(b * CPB + j, send_ref, recv2).wait_recv()
                return carry
            lax.fori_loop(0, CPB, step, 0)
        comm(drain)

        for h in range(H):
            qT = q_ref[0, h].astype(jnp.bfloat16)
            o_r, m_r, den_r = _half_head(qT, krbuf[b, h], vrbuf[b, h])
            m_l = mscr[b, h, 0:1, :]
            den_l = mscr[b, h, 1:2, :]
            m = jnp.maximum(m_l, m_r)
            a_l = jnp.exp(m_l - m)
            a_r = jnp.exp(m_r - m)
            den = den_l * a_l + den_r * a_r
            o = (olscr[b, h] * a_l + o_r * a_r) / den
            o_ref[0, h] = o.astype(jnp.bfloat16)

    @pl.when(i == 2 * B)
    def _():
        def drains(send_ref, recv1, recv2):
            for c in range(C):
                mk_p1(c, send_ref, recv1).wait_send()
                mk_fwd(c, recv1).wait_send()
        comm(drains)


def kernel(Q, K, V):
    QT = jnp.transpose(Q, (0, 2, 3, 1))
    KTb = jnp.transpose(K, (0, 2, 3, 1)).astype(jnp.bfloat16)
    VTb = jnp.transpose(V, (0, 2, 3, 1)).astype(jnp.bfloat16)

    def im_q(i):
        return (jnp.where(i >= B + 1, i - (B + 1), jnp.clip(i - 1, 0, B - 1)),
                0, 0, 0)

    def im_o(i):
        return (jnp.clip(i - (B + 1), 0, B - 1), 0, 0, 0)

    out = pl.pallas_call(
        _body,
        grid=(2 * B + 1,),
        in_specs=[
            pl.BlockSpec((1, H, D, SQ), im_q),
            pl.BlockSpec(memory_space=pltpu.VMEM),
            pl.BlockSpec(memory_space=pltpu.VMEM),
        ],
        out_specs=pl.BlockSpec((1, H, D, SQ), im_o),
        out_shape=jax.ShapeDtypeStruct((B, H, D, SQ), jnp.bfloat16),
        scratch_shapes=[
            pltpu.VMEM((B, H, D, SQ), jnp.bfloat16),
            pltpu.VMEM((B, H, D, SQ), jnp.bfloat16),
            pltpu.VMEM((B, H, D, SQ), jnp.float32),
            pltpu.VMEM((B, H, 8, SQ), jnp.float32),
            pltpu.SemaphoreType.DMA((C,)),
            pltpu.SemaphoreType.DMA((C,)),
            pltpu.SemaphoreType.DMA((C,)),
            pltpu.SemaphoreType.DMA((C,)),
            pltpu.SemaphoreType.REGULAR,
            pltpu.SemaphoreType.REGULAR,
        ],
        compiler_params=pltpu.CompilerParams(
            dimension_semantics=("arbitrary",)),
    )(QT, KTb, VTb)
    return out.transpose(0, 3, 1, 2).astype(jnp.float32)
